# Initial kernel scaffold; baseline (speedup 1.0000x reference)
import jax
import jax.numpy as jnp
from jax.experimental import pallas as pl


def kernel(x, w1, w2, wa, g1, b1, g2, b2, ga, ba):
    raise NotImplementedError("write your pallas kernel here")


# speedup vs baseline: 2.2040x; 2.2040x over previous
import jax
import jax.numpy as jnp
from jax.experimental import pallas as pl
from jax.experimental.pallas import tpu as pltpu

F32 = jnp.float32
BF16 = jnp.bfloat16
EPS = 1e-5
PADL = 128


def _fill9(p9_ref, padact_ref, C, H, W, L, M):
    pos = jax.lax.broadcasted_iota(jnp.int32, (1, L), 1)
    hh, ww = (pos // W) % H, pos % W
    t = 0
    for kh in range(3):
        for kw in range(3):
            dh, dw = kh - 1, kw - 1
            off = PADL + dh * W + dw
            seg = padact_ref[:, off:off + L]
            mask = None
            if dh != 0:
                mask = (hh + dh >= 0) & (hh + dh < H)
            if dw != 0:
                m = (ww + dw >= 0) & (ww + dw < W)
                mask = m if mask is None else (mask & m)
            if mask is not None:
                seg = jnp.where(mask, seg, jnp.zeros((), BF16))
            p9_ref[t * C:(t + 1) * C, M:M + L] = seg
            t += 1


def _conv9(p9_ref, w3_ref, HW, L):
    acc = None
    for kd in range(3):
        part = jax.lax.dot_general(
            w3_ref[kd], p9_ref[:, kd * HW:kd * HW + L],
            (((1,), (0,)), ((), ())), preferred_element_type=F32)
        acc = part if acc is None else acc + part
    return acc


def _stats(y, L):
    s = jnp.sum(y, axis=1, keepdims=True)
    q = jnp.sum((y - s * (1.0 / L)) ** 2, axis=1, keepdims=True)
    return s, q


def _bn_combine(s_p, q_p, L, N, g, b):
    total = float(L * N)
    mean = jnp.sum(s_p, axis=0) / total
    m_p = s_p * (1.0 / L)
    var = (jnp.sum(q_p, axis=0)
           + L * jnp.sum((m_p - mean) ** 2, axis=0)) / total
    scale = g * jax.lax.rsqrt(var + EPS)
    shift = b - mean * scale
    return scale, shift


def _make_p1(C_in, H, W, L):
    HW = H * W

    def body(x_ref, w13_ref, wa_ref, y1_ref, st_ref, padact_ref, p9_ref):
        act = x_ref[0].astype(BF16)
        padact_ref[:, 0:PADL] = jnp.zeros((C_in, PADL), BF16)
        padact_ref[:, PADL + L:] = jnp.zeros((C_in, PADL), BF16)
        padact_ref[:, PADL:PADL + L] = act
        p9_ref[:, 0:HW] = jnp.zeros((9 * C_in, HW), BF16)
        p9_ref[:, HW + L:] = jnp.zeros((9 * C_in, HW), BF16)
        _fill9(p9_ref, padact_ref, C_in, H, W, L, HW)
        y = _conv9(p9_ref, w13_ref, HW, L)
        y1_ref[0] = y.astype(BF16)
        s1, q1 = _stats(y, L)
        r = jax.lax.dot_general(wa_ref[...], act, (((1,), (0,)), ((), ())),
                                preferred_element_type=F32)
        sr, qr = _stats(r, L)
        st_ref[0] = jnp.concatenate([s1, q1, sr, qr], axis=1)
    return body


def _make_p2(C, H, W, L, N):
    HW = H * W

    def body(y1_ref, w23_ref, st1_ref, gb_ref, y2_ref, st_ref,
             padact_ref, p9_ref):
        st1 = st1_ref[...]
        scale1, shift1 = _bn_combine(st1[:, :, 0:1], st1[:, :, 1:2], L, N,
                                     gb_ref[:, 0:1], gb_ref[:, 1:2])
        a = jnp.maximum(y1_ref[0].astype(F32) * scale1 + shift1, 0.0)
        padact_ref[:, 0:PADL] = jnp.zeros((C, PADL), BF16)
        padact_ref[:, PADL + L:] = jnp.zeros((C, PADL), BF16)
        padact_ref[:, PADL:PADL + L] = a.astype(BF16)
        p9_ref[:, 0:HW] = jnp.zeros((9 * C, HW), BF16)
        p9_ref[:, HW + L:] = jnp.zeros((9 * C, HW), BF16)
        _fill9(p9_ref, padact_ref, C, H, W, L, HW)
        y = _conv9(p9_ref, w23_ref, HW, L)
        y2_ref[0] = y.astype(BF16)
        s2, q2 = _stats(y, L)
        st_ref[0] = jnp.concatenate([s2, q2], axis=1)
    return body


def _make_p3(L, N):
    def body(y2_ref, x_ref, wa_ref, st1_ref, st2_ref, gb_ref,
             out_ref, xbn_ref):
        st1, st2 = st1_ref[...], st2_ref[...]
        scale2, shift2 = _bn_combine(st2[:, :, 0:1], st2[:, :, 1:2], L, N,
                                     gb_ref[:, 2:3], gb_ref[:, 3:4])
        scale_r, shift_r = _bn_combine(st1[:, :, 2:3], st1[:, :, 3:4], L, N,
                                       gb_ref[:, 4:5], gb_ref[:, 5:6])
        xbn = y2_ref[0].astype(F32) * scale2 + shift2
        act = x_ref[0].astype(BF16)
        r = jax.lax.dot_general(wa_ref[...], act, (((1,), (0,)), ((), ())),
                                preferred_element_type=F32)
        out_ref[0] = jnp.maximum(xbn + (r * scale_r + shift_r), 0.0)
        xbn_ref[0] = xbn
    return body


def kernel(x, w1, w2, wa, g1, b1, g2, b2, ga, ba):
    N, C_in, D, H, W = x.shape
    C_out = w1.shape[0]
    L = D * H * W
    HW = H * W

    x_cf = x.reshape(N, C_in, L)
    w13 = w1.reshape(C_out, 3, 9 * C_in).swapaxes(0, 1).astype(BF16)
    w23 = w2.reshape(C_out, 3, 9 * C_out).swapaxes(0, 1).astype(BF16)
    wab = wa.astype(BF16)
    gb = jnp.stack([g1, b1, g2, b2, ga, ba], axis=1)

    par = pltpu.CompilerParams(dimension_semantics=("parallel",))
    y_spec = pl.BlockSpec((1, C_out, L), lambda n: (n, 0, 0))
    x_spec = pl.BlockSpec((1, C_in, L), lambda n: (n, 0, 0))
    gb_spec = pl.BlockSpec((C_out, 6), lambda n: (0, 0))

    y1, st1 = pl.pallas_call(
        _make_p1(C_in, H, W, L),
        out_shape=(jax.ShapeDtypeStruct((N, C_out, L), BF16),
                   jax.ShapeDtypeStruct((N, C_out, 4), F32)),
        grid=(N,),
        in_specs=[x_spec,
                  pl.BlockSpec((3, C_out, 9 * C_in), lambda n: (0, 0, 0)),
                  pl.BlockSpec((C_out, C_in), lambda n: (0, 0))],
        out_specs=(y_spec, pl.BlockSpec((1, C_out, 4), lambda n: (n, 0, 0))),
        scratch_shapes=[pltpu.VMEM((C_in, 2 * PADL + L), BF16),
                        pltpu.VMEM((9 * C_in, 2 * HW + L), BF16)],
        compiler_params=par,
    )(x_cf, w13, wab)

    st1_spec = pl.BlockSpec((N, C_out, 4), lambda n: (0, 0, 0))
    y2, st2 = pl.pallas_call(
        _make_p2(C_out, H, W, L, N),
        out_shape=(jax.ShapeDtypeStruct((N, C_out, L), BF16),
                   jax.ShapeDtypeStruct((N, C_out, 2), F32)),
        grid=(N,),
        in_specs=[y_spec,
                  pl.BlockSpec((3, C_out, 9 * C_out), lambda n: (0, 0, 0)),
                  st1_spec, gb_spec],
        out_specs=(y_spec, pl.BlockSpec((1, C_out, 2), lambda n: (n, 0, 0))),
        scratch_shapes=[pltpu.VMEM((C_out, 2 * PADL + L), BF16),
                        pltpu.VMEM((9 * C_out, 2 * HW + L), BF16)],
        compiler_params=par,
    )(y1, w23, st1, gb)

    out_cf, xbn_cf = pl.pallas_call(
        _make_p3(L, N),
        out_shape=(jax.ShapeDtypeStruct((N, C_out, L), F32),
                   jax.ShapeDtypeStruct((N, C_out, L), F32)),
        grid=(N,),
        in_specs=[y_spec, x_spec,
                  pl.BlockSpec((C_out, C_in), lambda n: (0, 0)),
                  st1_spec,
                  pl.BlockSpec((N, C_out, 2), lambda n: (0, 0, 0)),
                  gb_spec],
        out_specs=(pl.BlockSpec((1, C_out, L), lambda n: (n, 0, 0)),
                   pl.BlockSpec((1, C_out, L), lambda n: (n, 0, 0))),
        compiler_params=par,
    )(y2, x_cf, wab, st1, st2, gb)

    return (out_cf.reshape(N, C_out, D, H, W),
            xbn_cf.reshape(N, C_out, D, H, W))


# speedup vs baseline: 2.4470x; 1.1103x over previous
import jax
import jax.numpy as jnp
from jax.experimental import pallas as pl
from jax.experimental.pallas import tpu as pltpu

F32 = jnp.float32
BF16 = jnp.bfloat16
EPS = 1e-5
PADL = 128


def _fill9(p9_ref, padact_ref, C, H, W, L, M):
    pos = jax.lax.broadcasted_iota(jnp.int32, (1, L), 1)
    hh, ww = (pos // W) % H, pos % W
    t = 0
    for kh in range(3):
        for kw in range(3):
            dh, dw = kh - 1, kw - 1
            off = PADL + dh * W + dw
            seg = padact_ref[:, off:off + L]
            mask = None
            if dh != 0:
                mask = (hh + dh >= 0) & (hh + dh < H)
            if dw != 0:
                m = (ww + dw >= 0) & (ww + dw < W)
                mask = m if mask is None else (mask & m)
            if mask is not None:
                seg = jnp.where(mask, seg, jnp.zeros((), BF16))
            p9_ref[t * C:(t + 1) * C, M:M + L] = seg
            t += 1


def _conv9(p9_ref, w3_ref, HW, L):
    acc = None
    for kd in range(3):
        part = jax.lax.dot_general(
            w3_ref[kd], p9_ref[:, kd * HW:kd * HW + L],
            (((1,), (0,)), ((), ())), preferred_element_type=F32)
        acc = part if acc is None else acc + part
    return acc


def _stats(y, L):
    s = jnp.sum(y, axis=1, keepdims=True)
    q = jnp.sum((y - s * (1.0 / L)) ** 2, axis=1, keepdims=True)
    return s, q


def _bn_combine(s_p, q_p, L, g, b):
    N = s_p.shape[0]
    total = float(L * N)
    mean = jnp.sum(s_p, axis=0) / total
    m_p = s_p * (1.0 / L)
    var = (jnp.sum(q_p, axis=0)
           + L * jnp.sum((m_p - mean) ** 2, axis=0)) / total
    scale = g * jax.lax.rsqrt(var + EPS)
    shift = b - mean * scale
    return scale, shift


def _make_p1(C_in, H, W, L):
    HW = H * W

    def body(x_ref, w13_ref, wa_ref, y1_ref, st_ref, padact_ref, p9_ref):
        act = x_ref[0].astype(BF16)
        padact_ref[:, 0:PADL] = jnp.zeros((C_in, PADL), BF16)
        padact_ref[:, PADL + L:] = jnp.zeros((C_in, PADL), BF16)
        padact_ref[:, PADL:PADL + L] = act
        p9_ref[:, 0:HW] = jnp.zeros((9 * C_in, HW), BF16)
        p9_ref[:, HW + L:] = jnp.zeros((9 * C_in, HW), BF16)
        _fill9(p9_ref, padact_ref, C_in, H, W, L, HW)
        y = _conv9(p9_ref, w13_ref, HW, L)
        y1_ref[0] = y.astype(BF16)
        s1, q1 = _stats(y, L)
        r = jax.lax.dot_general(wa_ref[...], act, (((1,), (0,)), ((), ())),
                                preferred_element_type=F32)
        sr, qr = _stats(r, L)
        st_ref[0] = jnp.concatenate([s1, q1, sr, qr], axis=1)
    return body


def _make_p2(C, H, W, L):
    HW = H * W

    def body(y1_ref, w23_ref, ss1_ref, y2_ref, st_ref, padact_ref, p9_ref):
        scale1 = ss1_ref[:, 0:1].astype(BF16)
        shift1 = ss1_ref[:, 1:2].astype(BF16)
        a = jnp.maximum(y1_ref[0] * scale1 + shift1, jnp.zeros((), BF16))
        padact_ref[:, 0:PADL] = jnp.zeros((C, PADL), BF16)
        padact_ref[:, PADL + L:] = jnp.zeros((C, PADL), BF16)
        padact_ref[:, PADL:PADL + L] = a
        p9_ref[:, 0:HW] = jnp.zeros((9 * C, HW), BF16)
        p9_ref[:, HW + L:] = jnp.zeros((9 * C, HW), BF16)
        _fill9(p9_ref, padact_ref, C, H, W, L, HW)
        y = _conv9(p9_ref, w23_ref, HW, L)
        y2_ref[0] = y.astype(BF16)
        s2, q2 = _stats(y, L)
        st_ref[0] = jnp.concatenate([s2, q2], axis=1)
    return body


def _p3_body(y2_ref, x_ref, wa_ref, ssf_ref, out_ref, xbn_ref):
    xbn = y2_ref[0].astype(F32) * ssf_ref[:, 0:1] + ssf_ref[:, 1:2]
    act = x_ref[0].astype(BF16)
    r = jax.lax.dot_general(wa_ref[...], act, (((1,), (0,)), ((), ())),
                            preferred_element_type=F32)
    out_ref[0] = jnp.maximum(xbn + (r * ssf_ref[:, 2:3] + ssf_ref[:, 3:4]), 0.0)
    xbn_ref[0] = xbn


def kernel(x, w1, w2, wa, g1, b1, g2, b2, ga, ba):
    N, C_in, D, H, W = x.shape
    C_out = w1.shape[0]
    L = D * H * W
    HW = H * W

    x_cf = x.reshape(N, C_in, L)
    w13 = w1.reshape(C_out, 3, 9 * C_in).swapaxes(0, 1).astype(BF16)
    w23 = w2.reshape(C_out, 3, 9 * C_out).swapaxes(0, 1).astype(BF16)
    wab = wa.astype(BF16)

    par = pltpu.CompilerParams(dimension_semantics=("parallel",))
    y_spec = pl.BlockSpec((1, C_out, L), lambda n: (n, 0, 0))
    x_spec = pl.BlockSpec((1, C_in, L), lambda n: (n, 0, 0))

    y1, st1 = pl.pallas_call(
        _make_p1(C_in, H, W, L),
        out_shape=(jax.ShapeDtypeStruct((N, C_out, L), BF16),
                   jax.ShapeDtypeStruct((N, C_out, 4), F32)),
        grid=(N,),
        in_specs=[x_spec,
                  pl.BlockSpec((3, C_out, 9 * C_in), lambda n: (0, 0, 0)),
                  pl.BlockSpec((C_out, C_in), lambda n: (0, 0))],
        out_specs=(y_spec, pl.BlockSpec((1, C_out, 4), lambda n: (n, 0, 0))),
        scratch_shapes=[pltpu.VMEM((C_in, 2 * PADL + L), BF16),
                        pltpu.VMEM((9 * C_in, 2 * HW + L), BF16)],
        compiler_params=par,
    )(x_cf, w13, wab)

    scale1, shift1 = _bn_combine(st1[:, :, 0], st1[:, :, 1], L, g1, b1)
    ss1 = jnp.stack([scale1, shift1], axis=1)

    y2, st2 = pl.pallas_call(
        _make_p2(C_out, H, W, L),
        out_shape=(jax.ShapeDtypeStruct((N, C_out, L), BF16),
                   jax.ShapeDtypeStruct((N, C_out, 2), F32)),
        grid=(N,),
        in_specs=[y_spec,
                  pl.BlockSpec((3, C_out, 9 * C_out), lambda n: (0, 0, 0)),
                  pl.BlockSpec((C_out, 2), lambda n: (0, 0))],
        out_specs=(y_spec, pl.BlockSpec((1, C_out, 2), lambda n: (n, 0, 0))),
        scratch_shapes=[pltpu.VMEM((C_out, 2 * PADL + L), BF16),
                        pltpu.VMEM((9 * C_out, 2 * HW + L), BF16)],
        compiler_params=par,
    )(y1, w23, ss1)

    scale2, shift2 = _bn_combine(st2[:, :, 0], st2[:, :, 1], L, g2, b2)
    scale_r, shift_r = _bn_combine(st1[:, :, 2], st1[:, :, 3], L, ga, ba)
    ssf = jnp.stack([scale2, shift2, scale_r, shift_r], axis=1)

    out_cf, xbn_cf = pl.pallas_call(
        _p3_body,
        out_shape=(jax.ShapeDtypeStruct((N, C_out, L), F32),
                   jax.ShapeDtypeStruct((N, C_out, L), F32)),
        grid=(N,),
        in_specs=[y_spec, x_spec,
                  pl.BlockSpec((C_out, C_in), lambda n: (0, 0)),
                  pl.BlockSpec((C_out, 4), lambda n: (0, 0))],
        out_specs=(pl.BlockSpec((1, C_out, L), lambda n: (n, 0, 0)),
                   pl.BlockSpec((1, C_out, L), lambda n: (n, 0, 0))),
        compiler_params=par,
    )(y2, x_cf, wab, ssf)

    return (out_cf.reshape(N, C_out, D, H, W),
            xbn_cf.reshape(N, C_out, D, H, W))


# speedup vs baseline: 2.6028x; 1.0636x over previous
import jax
import jax.numpy as jnp
from jax.experimental import pallas as pl
from jax.experimental.pallas import tpu as pltpu

F32 = jnp.float32
BF16 = jnp.bfloat16
EPS = 1e-5
PADL = 128


def _fill9(p9_ref, padact_ref, C, H, W, Lw, M):
    pos = jax.lax.broadcasted_iota(jnp.int32, (1, Lw), 1)
    hh, ww = (pos // W) % H, pos % W
    t = 0
    for kh in range(3):
        for kw in range(3):
            dh, dw = kh - 1, kw - 1
            off = PADL + dh * W + dw
            seg = padact_ref[:, off:off + Lw]
            mask = None
            if dh != 0:
                mask = (hh + dh >= 0) & (hh + dh < H)
            if dw != 0:
                m = (ww + dw >= 0) & (ww + dw < W)
                mask = m if mask is None else (mask & m)
            if mask is not None:
                seg = jnp.where(mask, seg, jnp.zeros((), BF16))
            p9_ref[t * C:(t + 1) * C, M:M + Lw] = seg
            t += 1


def _conv9(p9_ref, w3_ref, HW, Lw):
    acc = None
    for kd in range(3):
        part = jax.lax.dot_general(
            w3_ref[kd], p9_ref[:, kd * HW:kd * HW + Lw],
            (((1,), (0,)), ((), ())), preferred_element_type=F32)
        acc = part if acc is None else acc + part
    return acc


def _stats(y, L):
    s = jnp.sum(y, axis=1, keepdims=True)
    q = jnp.sum((y - s * (1.0 / L)) ** 2, axis=1, keepdims=True)
    return s, q


def _bn_combine(s_p, q_p, L, g, b):
    N = s_p.shape[0]
    total = float(L * N)
    mean = jnp.sum(s_p, axis=0) / total
    m_p = s_p * (1.0 / L)
    var = (jnp.sum(q_p, axis=0)
           + L * jnp.sum((m_p - mean) ** 2, axis=0)) / total
    scale = g * jax.lax.rsqrt(var + EPS)
    shift = b - mean * scale
    return scale, shift


def _zero_pads(padact_ref, p9_ref, C, C9, HW, L, Lw):
    padact_ref[:, 0:PADL] = jnp.zeros((C, PADL), BF16)
    padact_ref[:, PADL + L:PADL + L + 2 * HW] = jnp.zeros((C, 2 * HW), BF16)
    padact_ref[:, PADL + Lw:] = jnp.zeros((C, PADL), BF16)
    p9_ref[:, 0:HW] = jnp.zeros((C9, HW), BF16)
    p9_ref[:, HW + Lw:] = jnp.zeros((C9, HW), BF16)


def _make_p1(C_in, H, W, L):
    HW = H * W
    S1 = L + 2 * HW
    Lw = L + 2 * HW + L

    def body(x_ref, w13_ref, wa_ref, y1_ref, st_ref, padact_ref, p9_ref):
        a0 = x_ref[0].astype(BF16)
        a1 = x_ref[1].astype(BF16)
        _zero_pads(padact_ref, p9_ref, C_in, 9 * C_in, HW, L, Lw)
        padact_ref[:, PADL:PADL + L] = a0
        padact_ref[:, PADL + S1:PADL + S1 + L] = a1
        _fill9(p9_ref, padact_ref, C_in, H, W, Lw, HW)
        y = _conv9(p9_ref, w13_ref, HW, Lw)
        y0, y1v = y[:, 0:L], y[:, S1:S1 + L]
        y1_ref[0] = y0.astype(BF16)
        y1_ref[1] = y1v.astype(BF16)
        r = jax.lax.dot_general(
            wa_ref[...], padact_ref[:, PADL:PADL + Lw],
            (((1,), (0,)), ((), ())), preferred_element_type=F32)
        for i, (yv, rv) in enumerate(((y0, r[:, 0:L]), (y1v, r[:, S1:S1 + L]))):
            s1, q1 = _stats(yv, L)
            sr, qr = _stats(rv, L)
            st_ref[i] = jnp.concatenate([s1, q1, sr, qr], axis=1)
    return body


def _make_p2(C, H, W, L):
    HW = H * W
    S1 = L + 2 * HW
    Lw = L + 2 * HW + L

    def body(y1_ref, w23_ref, ss1_ref, y2_ref, st_ref, padact_ref, p9_ref):
        scale1 = ss1_ref[:, 0:1].astype(BF16)
        shift1 = ss1_ref[:, 1:2].astype(BF16)
        zero = jnp.zeros((), BF16)
        _zero_pads(padact_ref, p9_ref, C, 9 * C, HW, L, Lw)
        padact_ref[:, PADL:PADL + L] = jnp.maximum(
            y1_ref[0] * scale1 + shift1, zero)
        padact_ref[:, PADL + S1:PADL + S1 + L] = jnp.maximum(
            y1_ref[1] * scale1 + shift1, zero)
        _fill9(p9_ref, padact_ref, C, H, W, Lw, HW)
        y = _conv9(p9_ref, w23_ref, HW, Lw)
        y0, y1v = y[:, 0:L], y[:, S1:S1 + L]
        y2_ref[0] = y0.astype(BF16)
        y2_ref[1] = y1v.astype(BF16)
        for i, yv in enumerate((y0, y1v)):
            s2, q2 = _stats(yv, L)
            st_ref[i] = jnp.concatenate([s2, q2], axis=1)
    return body


def _p3_body(y2_ref, x_ref, wa_ref, ssf_ref, out_ref, xbn_ref):
    scale2, shift2 = ssf_ref[:, 0:1], ssf_ref[:, 1:2]
    scale_r, shift_r = ssf_ref[:, 2:3], ssf_ref[:, 3:4]
    for i in range(2):
        xbn = y2_ref[i].astype(F32) * scale2 + shift2
        act = x_ref[i].astype(BF16)
        r = jax.lax.dot_general(wa_ref[...], act, (((1,), (0,)), ((), ())),
                                preferred_element_type=F32)
        out_ref[i] = jnp.maximum(xbn + (r * scale_r + shift_r), 0.0)
        xbn_ref[i] = xbn


def kernel(x, w1, w2, wa, g1, b1, g2, b2, ga, ba):
    N, C_in, D, H, W = x.shape
    C_out = w1.shape[0]
    L = D * H * W
    HW = H * W
    Lw = 2 * L + 2 * HW
    B = 2
    G = N // B

    x_cf = x.reshape(N, C_in, L)
    w13 = w1.reshape(C_out, 3, 9 * C_in).swapaxes(0, 1).astype(BF16)
    w23 = w2.reshape(C_out, 3, 9 * C_out).swapaxes(0, 1).astype(BF16)
    wab = wa.astype(BF16)

    par = pltpu.CompilerParams(dimension_semantics=("parallel",))
    y_spec = pl.BlockSpec((B, C_out, L), lambda n: (n, 0, 0))
    x_spec = pl.BlockSpec((B, C_in, L), lambda n: (n, 0, 0))

    y1, st1 = pl.pallas_call(
        _make_p1(C_in, H, W, L),
        out_shape=(jax.ShapeDtypeStruct((N, C_out, L), BF16),
                   jax.ShapeDtypeStruct((N, C_out, 4), F32)),
        grid=(G,),
        in_specs=[x_spec,
                  pl.BlockSpec((3, C_out, 9 * C_in), lambda n: (0, 0, 0)),
                  pl.BlockSpec((C_out, C_in), lambda n: (0, 0))],
        out_specs=(y_spec, pl.BlockSpec((B, C_out, 4), lambda n: (n, 0, 0))),
        scratch_shapes=[pltpu.VMEM((C_in, 2 * PADL + Lw), BF16),
                        pltpu.VMEM((9 * C_in, 2 * HW + Lw), BF16)],
        compiler_params=par,
    )(x_cf, w13, wab)

    scale1, shift1 = _bn_combine(st1[:, :, 0], st1[:, :, 1], L, g1, b1)
    ss1 = jnp.stack([scale1, shift1], axis=1)

    y2, st2 = pl.pallas_call(
        _make_p2(C_out, H, W, L),
        out_shape=(jax.ShapeDtypeStruct((N, C_out, L), BF16),
                   jax.ShapeDtypeStruct((N, C_out, 2), F32)),
        grid=(G,),
        in_specs=[y_spec,
                  pl.BlockSpec((3, C_out, 9 * C_out), lambda n: (0, 0, 0)),
                  pl.BlockSpec((C_out, 2), lambda n: (0, 0))],
        out_specs=(y_spec, pl.BlockSpec((B, C_out, 2), lambda n: (n, 0, 0))),
        scratch_shapes=[pltpu.VMEM((C_out, 2 * PADL + Lw), BF16),
                        pltpu.VMEM((9 * C_out, 2 * HW + Lw), BF16)],
        compiler_params=par,
    )(y1, w23, ss1)

    scale2, shift2 = _bn_combine(st2[:, :, 0], st2[:, :, 1], L, g2, b2)
    scale_r, shift_r = _bn_combine(st1[:, :, 2], st1[:, :, 3], L, ga, ba)
    ssf = jnp.stack([scale2, shift2, scale_r, shift_r], axis=1)

    out_cf, xbn_cf = pl.pallas_call(
        _p3_body,
        out_shape=(jax.ShapeDtypeStruct((N, C_out, L), F32),
                   jax.ShapeDtypeStruct((N, C_out, L), F32)),
        grid=(G,),
        in_specs=[y_spec, x_spec,
                  pl.BlockSpec((C_out, C_in), lambda n: (0, 0)),
                  pl.BlockSpec((C_out, 4), lambda n: (0, 0))],
        out_specs=(pl.BlockSpec((B, C_out, L), lambda n: (n, 0, 0)),
                   pl.BlockSpec((B, C_out, L), lambda n: (n, 0, 0))),
        compiler_params=par,
    )(y2, x_cf, wab, ssf)

    return (out_cf.reshape(N, C_out, D, H, W),
            xbn_cf.reshape(N, C_out, D, H, W))


# speedup vs baseline: 2.8010x; 1.0761x over previous
import jax
import jax.numpy as jnp
from jax.experimental import pallas as pl
from jax.experimental.pallas import tpu as pltpu

F32 = jnp.float32
BF16 = jnp.bfloat16
EPS = 1e-5
PADL = 128


def _fill9(p9_ref, padact_ref, C, H, W, L, pa_off, p9_off):
    pos = jax.lax.broadcasted_iota(jnp.int32, (1, L), 1)
    hh, ww = (pos // W) % H, pos % W
    t = 0
    for kh in range(3):
        for kw in range(3):
            dh, dw = kh - 1, kw - 1
            off = pa_off + dh * W + dw
            seg = padact_ref[:, off:off + L]
            mask = None
            if dh != 0:
                mask = (hh + dh >= 0) & (hh + dh < H)
            if dw != 0:
                m = (ww + dw >= 0) & (ww + dw < W)
                mask = m if mask is None else (mask & m)
            if mask is not None:
                seg = jnp.where(mask, seg, jnp.zeros((), BF16))
            p9_ref[t * C:(t + 1) * C, p9_off:p9_off + L] = seg
            t += 1


def _conv9(p9_ref, w3_ref, HW, L, base):
    acc = None
    for kd in range(3):
        part = jax.lax.dot_general(
            w3_ref[kd], p9_ref[:, base + kd * HW:base + kd * HW + L],
            (((1,), (0,)), ((), ())), preferred_element_type=F32)
        acc = part if acc is None else acc + part
    return acc


def _stats(y):
    s = jnp.sum(y, axis=1, keepdims=True)
    q = jnp.sum(y * y, axis=1, keepdims=True)
    return s, q


def _bn_combine(s_p, q_p, L, g, b):
    N = s_p.shape[0]
    total = float(L * N)
    mean = jnp.sum(s_p, axis=0) / total
    var = jnp.sum(q_p, axis=0) / total - mean * mean
    scale = g * jax.lax.rsqrt(var + EPS)
    shift = b - mean * scale
    return scale, shift


def _zero_pads(padact_ref, p9_ref, C, C9, HW, L, Lw, S1):
    padact_ref[:, 0:PADL] = jnp.zeros((C, PADL), BF16)
    padact_ref[:, PADL + L:PADL + L + 2 * HW] = jnp.zeros((C, 2 * HW), BF16)
    padact_ref[:, PADL + Lw:] = jnp.zeros((C, PADL), BF16)
    p9_ref[:, 0:HW] = jnp.zeros((C9, HW), BF16)
    p9_ref[:, HW + L:HW + S1] = jnp.zeros((C9, 2 * HW), BF16)
    p9_ref[:, HW + Lw:] = jnp.zeros((C9, HW), BF16)


def _make_p1(C_in, H, W, L):
    HW = H * W
    S1 = L + 2 * HW
    Lw = L + 2 * HW + L

    def body(x_ref, w13_ref, wa_ref, y1_ref, st_ref, padact_ref, p9_ref):
        _zero_pads(padact_ref, p9_ref, C_in, 9 * C_in, HW, L, Lw, S1)
        ys = []
        for i in range(2):
            padact_ref[:, PADL + i * S1:PADL + i * S1 + L] = \
                x_ref[i].astype(BF16)
            _fill9(p9_ref, padact_ref, C_in, H, W, L,
                   PADL + i * S1, HW + i * S1)
            ys.append(_conv9(p9_ref, w13_ref, HW, L, i * S1))
        r = jax.lax.dot_general(
            wa_ref[...], padact_ref[:, PADL:PADL + Lw],
            (((1,), (0,)), ((), ())), preferred_element_type=F32)
        for i, yv in enumerate(ys):
            y1_ref[i] = yv.astype(BF16)
            s1, q1 = _stats(yv)
            sr, qr = _stats(r[:, i * S1:i * S1 + L])
            st_ref[i] = jnp.concatenate([s1, q1, sr, qr], axis=1)
    return body


def _make_p2(C, H, W, L):
    HW = H * W
    S1 = L + 2 * HW
    Lw = L + 2 * HW + L

    def body(y1_ref, w23_ref, ss1_ref, y2_ref, st_ref, padact_ref, p9_ref):
        scale1 = ss1_ref[:, 0:1].astype(BF16)
        shift1 = ss1_ref[:, 1:2].astype(BF16)
        zero = jnp.zeros((), BF16)
        _zero_pads(padact_ref, p9_ref, C, 9 * C, HW, L, Lw, S1)
        ys = []
        for i in range(2):
            padact_ref[:, PADL + i * S1:PADL + i * S1 + L] = jnp.maximum(
                y1_ref[i] * scale1 + shift1, zero)
            _fill9(p9_ref, padact_ref, C, H, W, L,
                   PADL + i * S1, HW + i * S1)
            ys.append(_conv9(p9_ref, w23_ref, HW, L, i * S1))
        for i, yv in enumerate(ys):
            y2_ref[i] = yv.astype(BF16)
            s2, q2 = _stats(yv)
            st_ref[i] = jnp.concatenate([s2, q2], axis=1)
    return body


def _p3_body(y2_ref, x_ref, wa_ref, ssf_ref, out_ref, xbn_ref):
    scale2, shift2 = ssf_ref[:, 0:1], ssf_ref[:, 1:2]
    scale_r, shift_r = ssf_ref[:, 2:3], ssf_ref[:, 3:4]
    for i in range(2):
        xbn = y2_ref[i].astype(F32) * scale2 + shift2
        act = x_ref[i].astype(BF16)
        r = jax.lax.dot_general(wa_ref[...], act, (((1,), (0,)), ((), ())),
                                preferred_element_type=F32)
        out_ref[i] = jnp.maximum(xbn + (r * scale_r + shift_r), 0.0)
        xbn_ref[i] = xbn


def kernel(x, w1, w2, wa, g1, b1, g2, b2, ga, ba):
    N, C_in, D, H, W = x.shape
    C_out = w1.shape[0]
    L = D * H * W
    HW = H * W
    Lw = 2 * L + 2 * HW
    B = 2
    G = N // B

    x_cf = x.reshape(N, C_in, L)
    w13 = w1.reshape(C_out, 3, 9 * C_in).swapaxes(0, 1).astype(BF16)
    w23 = w2.reshape(C_out, 3, 9 * C_out).swapaxes(0, 1).astype(BF16)
    wab = wa.astype(BF16)

    par = pltpu.CompilerParams(dimension_semantics=("parallel",))
    y_spec = pl.BlockSpec((B, C_out, L), lambda n: (n, 0, 0))
    x_spec = pl.BlockSpec((B, C_in, L), lambda n: (n, 0, 0))

    y1, st1 = pl.pallas_call(
        _make_p1(C_in, H, W, L),
        out_shape=(jax.ShapeDtypeStruct((N, C_out, L), BF16),
                   jax.ShapeDtypeStruct((N, C_out, 4), F32)),
        grid=(G,),
        in_specs=[x_spec,
                  pl.BlockSpec((3, C_out, 9 * C_in), lambda n: (0, 0, 0)),
                  pl.BlockSpec((C_out, C_in), lambda n: (0, 0))],
        out_specs=(y_spec, pl.BlockSpec((B, C_out, 4), lambda n: (n, 0, 0))),
        scratch_shapes=[pltpu.VMEM((C_in, 2 * PADL + Lw), BF16),
                        pltpu.VMEM((9 * C_in, 2 * HW + Lw), BF16)],
        compiler_params=par,
    )(x_cf, w13, wab)

    scale1, shift1 = _bn_combine(st1[:, :, 0], st1[:, :, 1], L, g1, b1)
    ss1 = jnp.stack([scale1, shift1], axis=1)

    y2, st2 = pl.pallas_call(
        _make_p2(C_out, H, W, L),
        out_shape=(jax.ShapeDtypeStruct((N, C_out, L), BF16),
                   jax.ShapeDtypeStruct((N, C_out, 2), F32)),
        grid=(G,),
        in_specs=[y_spec,
                  pl.BlockSpec((3, C_out, 9 * C_out), lambda n: (0, 0, 0)),
                  pl.BlockSpec((C_out, 2), lambda n: (0, 0))],
        out_specs=(y_spec, pl.BlockSpec((B, C_out, 2), lambda n: (n, 0, 0))),
        scratch_shapes=[pltpu.VMEM((C_out, 2 * PADL + Lw), BF16),
                        pltpu.VMEM((9 * C_out, 2 * HW + Lw), BF16)],
        compiler_params=par,
    )(y1, w23, ss1)

    scale2, shift2 = _bn_combine(st2[:, :, 0], st2[:, :, 1], L, g2, b2)
    scale_r, shift_r = _bn_combine(st1[:, :, 2], st1[:, :, 3], L, ga, ba)
    ssf = jnp.stack([scale2, shift2, scale_r, shift_r], axis=1)

    out_cf, xbn_cf = pl.pallas_call(
        _p3_body,
        out_shape=(jax.ShapeDtypeStruct((N, C_out, L), F32),
                   jax.ShapeDtypeStruct((N, C_out, L), F32)),
        grid=(G,),
        in_specs=[y_spec, x_spec,
                  pl.BlockSpec((C_out, C_in), lambda n: (0, 0)),
                  pl.BlockSpec((C_out, 4), lambda n: (0, 0))],
        out_specs=(pl.BlockSpec((B, C_out, L), lambda n: (n, 0, 0)),
                   pl.BlockSpec((B, C_out, L), lambda n: (n, 0, 0))),
        compiler_params=par,
    )(y2, x_cf, wab, ssf)

    return (out_cf.reshape(N, C_out, D, H, W),
            xbn_cf.reshape(N, C_out, D, H, W))


# speedup vs baseline: 2.9193x; 1.0423x over previous
import jax
import jax.numpy as jnp
from jax.experimental import pallas as pl
from jax.experimental.pallas import tpu as pltpu

F32 = jnp.float32
BF16 = jnp.bfloat16
EPS = 1e-5
PADL = 128


def _fill9(p9_ref, padact_ref, C, H, W, L, pa_off, p9_off):
    pos = jax.lax.broadcasted_iota(jnp.int32, (1, L), 1)
    hh, ww = (pos // W) % H, pos % W
    t = 0
    for kh in range(3):
        for kw in range(3):
            dh, dw = kh - 1, kw - 1
            off = pa_off + dh * W + dw
            seg = padact_ref[:C, off:off + L]
            mask = None
            if dh != 0:
                mask = (hh + dh >= 0) & (hh + dh < H)
            if dw != 0:
                m = (ww + dw >= 0) & (ww + dw < W)
                mask = m if mask is None else (mask & m)
            if mask is not None:
                seg = jnp.where(mask, seg, jnp.zeros((), BF16))
            p9_ref[t * C:(t + 1) * C, p9_off:p9_off + L] = seg
            t += 1


def _conv9(p9_ref, w3_ref, C9, HW, L, base):
    acc = None
    for kd in range(3):
        part = jax.lax.dot_general(
            w3_ref[kd][:, :C9], p9_ref[:C9, base + kd * HW:base + kd * HW + L],
            (((1,), (0,)), ((), ())), preferred_element_type=F32)
        acc = part if acc is None else acc + part
    return acc


def _stats(y):
    s = jnp.sum(y, axis=1, keepdims=True)
    q = jnp.sum(y * y, axis=1, keepdims=True)
    return s, q


def _scale_shift(s, q, total, g, b):
    mean = s * (1.0 / total)
    var = q * (1.0 / total) - mean * mean
    scale = g * jax.lax.rsqrt(var + EPS)
    shift = b - mean * scale
    return scale, shift


def _make_body(N, C_in, C_out, D, H, W):
    L = D * H * W
    HW = H * W
    S1 = L + 2 * HW
    Lw = L + 2 * HW + L
    B = 2
    G = N // B
    total = float(N * L)

    def body(x_ref, w13_ref, w23_ref, wa_ref, gb_ref, out_ref, xbn_ref,
             ystore_ref, padact_ref, p9_ref, stacc_ref, sscr_ref):
        g = pl.program_id(0)

        @pl.when(g == 0)
        def _init():
            stacc_ref[...] = jnp.zeros_like(stacc_ref)
            padact_ref[:, 0:PADL] = jnp.zeros((C_out, PADL), BF16)
            padact_ref[:, PADL + L:PADL + L + 2 * HW] = (
                jnp.zeros((C_out, 2 * HW), BF16))
            padact_ref[:, PADL + Lw:] = jnp.zeros((C_out, PADL), BF16)
            p9_ref[:, 0:HW] = jnp.zeros((9 * C_out, HW), BF16)
            p9_ref[:, HW + L:HW + S1] = jnp.zeros((9 * C_out, 2 * HW), BF16)
            p9_ref[:, HW + Lw:] = jnp.zeros((9 * C_out, HW), BF16)

        @pl.when(g < G)
        def _phase_a():
            row = (g * B) * C_out
            ys = []
            for i in range(2):
                padact_ref[:C_in, PADL + i * S1:PADL + i * S1 + L] = \
                    x_ref[i].astype(BF16)
                _fill9(p9_ref, padact_ref, C_in, H, W, L,
                       PADL + i * S1, HW + i * S1)
                ys.append(_conv9(p9_ref, w13_ref, 9 * C_in, HW, L, i * S1))
            r = jax.lax.dot_general(
                wa_ref[...], padact_ref[:C_in, PADL:PADL + Lw],
                (((1,), (0,)), ((), ())), preferred_element_type=F32)
            ds = dq = dsr = dqr = None
            for i, yv in enumerate(ys):
                ystore_ref[pl.ds(row + i * C_out, C_out), :] = yv.astype(BF16)
                s1, q1 = _stats(yv)
                sr, qr = _stats(r[:, i * S1:i * S1 + L])
                ds = s1 if ds is None else ds + s1
                dq = q1 if dq is None else dq + q1
                dsr = sr if dsr is None else dsr + sr
                dqr = qr if dqr is None else dqr + qr
            stacc_ref[:, 0:4] = (stacc_ref[:, 0:4]
                                 + jnp.concatenate([ds, dq, dsr, dqr], axis=1))

        @pl.when(g == G)
        def _combine1():
            scale1, shift1 = _scale_shift(
                stacc_ref[:, 0:1], stacc_ref[:, 1:2], total,
                gb_ref[:, 0:1], gb_ref[:, 1:2])
            sscr_ref[:, 0:1] = scale1
            sscr_ref[:, 1:2] = shift1

        @pl.when((g >= G) & (g < 2 * G))
        def _phase_b():
            row = ((g - G) * B) * C_out
            scale1 = sscr_ref[:, 0:1].astype(BF16)
            shift1 = sscr_ref[:, 1:2].astype(BF16)
            zero = jnp.zeros((), BF16)
            ys = []
            for i in range(2):
                padact_ref[:, PADL + i * S1:PADL + i * S1 + L] = jnp.maximum(
                    ystore_ref[pl.ds(row + i * C_out, C_out), :]
                    * scale1 + shift1, zero)
                _fill9(p9_ref, padact_ref, C_out, H, W, L,
                       PADL + i * S1, HW + i * S1)
                ys.append(_conv9(p9_ref, w23_ref, 9 * C_out, HW, L, i * S1))
            ds = dq = None
            for i, yv in enumerate(ys):
                ystore_ref[pl.ds(row + i * C_out, C_out), :] = yv.astype(BF16)
                s2, q2 = _stats(yv)
                ds = s2 if ds is None else ds + s2
                dq = q2 if dq is None else dq + q2
            stacc_ref[:, 4:6] = stacc_ref[:, 4:6] + jnp.concatenate(
                [ds, dq], axis=1)

        @pl.when(g == 2 * G)
        def _combine2():
            scale2, shift2 = _scale_shift(
                stacc_ref[:, 4:5], stacc_ref[:, 5:6], total,
                gb_ref[:, 2:3], gb_ref[:, 3:4])
            scale_r, shift_r = _scale_shift(
                stacc_ref[:, 2:3], stacc_ref[:, 3:4], total,
                gb_ref[:, 4:5], gb_ref[:, 5:6])
            sscr_ref[:, 2:3] = scale2
            sscr_ref[:, 3:4] = shift2
            sscr_ref[:, 4:5] = scale_r
            sscr_ref[:, 5:6] = shift_r

        @pl.when(g >= 2 * G)
        def _phase_c():
            row = ((g - 2 * G) * B) * C_out
            scale2, shift2 = sscr_ref[:, 2:3], sscr_ref[:, 3:4]
            scale_r, shift_r = sscr_ref[:, 4:5], sscr_ref[:, 5:6]
            for i in range(2):
                y2 = ystore_ref[pl.ds(row + i * C_out, C_out), :]
                xbn = y2.astype(F32) * scale2 + shift2
                act = x_ref[i].astype(BF16)
                r = jax.lax.dot_general(
                    wa_ref[...], act, (((1,), (0,)), ((), ())),
                    preferred_element_type=F32)
                out_ref[i] = jnp.maximum(xbn + (r * scale_r + shift_r), 0.0)
                xbn_ref[i] = xbn
    return body


def kernel(x, w1, w2, wa, g1, b1, g2, b2, ga, ba):
    N, C_in, D, H, W = x.shape
    C_out = w1.shape[0]
    L = D * H * W
    HW = H * W
    Lw = 2 * L + 2 * HW
    B = 2
    G = N // B

    x_cf = x.reshape(N, C_in, L)
    w13 = w1.reshape(C_out, 3, 9 * C_in).swapaxes(0, 1).astype(BF16)
    w23 = w2.reshape(C_out, 3, 9 * C_out).swapaxes(0, 1).astype(BF16)
    wab = wa.astype(BF16)
    gb = jnp.stack([g1, b1, g2, b2, ga, ba], axis=1)

    def x_idx(g):
        return (jnp.where(g < G, g, jnp.where(g < 2 * G, G - 1, g - 2 * G)),
                0, 0)

    def out_idx(g):
        return (jnp.maximum(g - 2 * G, 0), 0, 0)

    out_cf, xbn_cf = pl.pallas_call(
        _make_body(N, C_in, C_out, D, H, W),
        out_shape=(jax.ShapeDtypeStruct((N, C_out, L), F32),
                   jax.ShapeDtypeStruct((N, C_out, L), F32)),
        grid=(3 * G,),
        in_specs=[pl.BlockSpec((B, C_in, L), x_idx),
                  pl.BlockSpec((3, C_out, 9 * C_in), lambda g: (0, 0, 0)),
                  pl.BlockSpec((3, C_out, 9 * C_out), lambda g: (0, 0, 0)),
                  pl.BlockSpec((C_out, C_in), lambda g: (0, 0)),
                  pl.BlockSpec((C_out, 6), lambda g: (0, 0))],
        out_specs=(pl.BlockSpec((B, C_out, L), out_idx),
                   pl.BlockSpec((B, C_out, L), out_idx)),
        scratch_shapes=[pltpu.VMEM((N * C_out, L), BF16),
                        pltpu.VMEM((C_out, 2 * PADL + Lw), BF16),
                        pltpu.VMEM((9 * C_out, 2 * HW + Lw), BF16),
                        pltpu.VMEM((C_out, 8), F32),
                        pltpu.VMEM((C_out, 8), F32)],
        compiler_params=pltpu.CompilerParams(
            dimension_semantics=("arbitrary",)),
    )(x_cf, w13, w23, wab, gb)

    return (out_cf.reshape(N, C_out, D, H, W),
            xbn_cf.reshape(N, C_out, D, H, W))


# speedup vs baseline: 3.5634x; 1.2206x over previous
import jax
import jax.numpy as jnp
from jax.experimental import pallas as pl
from jax.experimental.pallas import tpu as pltpu

F32 = jnp.float32
BF16 = jnp.bfloat16
EPS = 1e-5
PADL = 128


def _fill9(p9_ref, padact_ref, C, H, W, L, pa_off, p9_off):
    pos = jax.lax.broadcasted_iota(jnp.int32, (1, L), 1)
    hh, ww = (pos // W) % H, pos % W
    t = 0
    for kh in range(3):
        for kw in range(3):
            dh, dw = kh - 1, kw - 1
            off = pa_off + dh * W + dw
            seg = padact_ref[:C, off:off + L]
            mask = None
            if dh != 0:
                mask = (hh + dh >= 0) & (hh + dh < H)
            if dw != 0:
                m = (ww + dw >= 0) & (ww + dw < W)
                mask = m if mask is None else (mask & m)
            if mask is not None:
                seg = jnp.where(mask, seg, jnp.zeros((), BF16))
            p9_ref[t * C:(t + 1) * C, p9_off:p9_off + L] = seg
            t += 1


def _conv9(p9_ref, w3_ref, C9, HW, L, base):
    acc = None
    for kd in range(3):
        part = jax.lax.dot_general(
            w3_ref[kd][:, :C9], p9_ref[:C9, base + kd * HW:base + kd * HW + L],
            (((1,), (0,)), ((), ())), preferred_element_type=F32)
        acc = part if acc is None else acc + part
    return acc


def _stats(y):
    s = jnp.sum(y, axis=1, keepdims=True)
    q = jnp.sum(y * y, axis=1, keepdims=True)
    return s, q


def _scale_shift(s, q, total, g, b):
    mean = s * (1.0 / total)
    var = q * (1.0 / total) - mean * mean
    scale = g * jax.lax.rsqrt(var + EPS)
    shift = b - mean * scale
    return scale, shift


def _make_body(N, C_in, C_out, D, H, W):
    L = D * H * W
    HW = H * W
    S1 = L + 2 * HW
    Lw = L + 2 * HW + L
    B = 2
    G = N // B
    total = float(N * L)

    def body(x_ref, w13_ref, w23_ref, wa_ref, gb_ref, out_ref, xbn_ref,
             ystore_ref, padact_ref, p9_ref, stacc_ref, sscr_ref):
        g = pl.program_id(0)

        @pl.when(g == 0)
        def _init():
            stacc_ref[...] = jnp.zeros_like(stacc_ref)
            padact_ref[:, 0:PADL] = jnp.zeros((C_out, PADL), BF16)
            padact_ref[:, PADL + L:PADL + L + 2 * HW] = (
                jnp.zeros((C_out, 2 * HW), BF16))
            padact_ref[:, PADL + Lw:] = jnp.zeros((C_out, PADL), BF16)
            p9_ref[:, 0:HW] = jnp.zeros((9 * C_out, HW), BF16)
            p9_ref[:, HW + L:HW + S1] = jnp.zeros((9 * C_out, 2 * HW), BF16)
            p9_ref[:, HW + Lw:] = jnp.zeros((9 * C_out, HW), BF16)

        @pl.when(g < G)
        def _phase_a():
            row = (g * B) * C_out
            ys = []
            for i in range(2):
                padact_ref[:C_in, PADL + i * S1:PADL + i * S1 + L] = \
                    x_ref[i].astype(BF16)
                _fill9(p9_ref, padact_ref, C_in, H, W, L,
                       PADL + i * S1, HW + i * S1)
                ys.append(_conv9(p9_ref, w13_ref, 9 * C_in, HW, L, i * S1))
            r = jax.lax.dot_general(
                wa_ref[...], padact_ref[:C_in, PADL:PADL + Lw],
                (((1,), (0,)), ((), ())), preferred_element_type=F32)
            ds = dq = dsr = dqr = None
            for i, yv in enumerate(ys):
                ystore_ref[pl.ds(row + i * C_out, C_out), :] = yv.astype(BF16)
                s1, q1 = _stats(yv)
                sr, qr = _stats(r[:, i * S1:i * S1 + L])
                ds = s1 if ds is None else ds + s1
                dq = q1 if dq is None else dq + q1
                dsr = sr if dsr is None else dsr + sr
                dqr = qr if dqr is None else dqr + qr
            stacc_ref[:, 0:4] = (stacc_ref[:, 0:4]
                                 + jnp.concatenate([ds, dq, dsr, dqr], axis=1))

        @pl.when(g == G)
        def _combine1():
            scale1, shift1 = _scale_shift(
                stacc_ref[:, 0:1], stacc_ref[:, 1:2], total,
                gb_ref[:, 0:1], gb_ref[:, 1:2])
            sscr_ref[:, 0:1] = scale1
            sscr_ref[:, 1:2] = shift1

        @pl.when((g >= G) & (g < 2 * G))
        def _phase_b():
            row = ((g - G) * B) * C_out
            scale1 = sscr_ref[:, 0:1].astype(BF16)
            shift1 = sscr_ref[:, 1:2].astype(BF16)
            zero = jnp.zeros((), BF16)
            ys = []
            for i in range(2):
                padact_ref[:, PADL + i * S1:PADL + i * S1 + L] = jnp.maximum(
                    ystore_ref[pl.ds(row + i * C_out, C_out), :]
                    * scale1 + shift1, zero)
                _fill9(p9_ref, padact_ref, C_out, H, W, L,
                       PADL + i * S1, HW + i * S1)
                ys.append(_conv9(p9_ref, w23_ref, 9 * C_out, HW, L, i * S1))
            ds = dq = None
            for i, yv in enumerate(ys):
                ystore_ref[pl.ds(row + i * C_out, C_out), :] = yv.astype(BF16)
                s2, q2 = _stats(yv)
                ds = s2 if ds is None else ds + s2
                dq = q2 if dq is None else dq + q2
            stacc_ref[:, 4:6] = stacc_ref[:, 4:6] + jnp.concatenate(
                [ds, dq], axis=1)

        @pl.when(g == 2 * G)
        def _combine2():
            scale2, shift2 = _scale_shift(
                stacc_ref[:, 4:5], stacc_ref[:, 5:6], total,
                gb_ref[:, 2:3], gb_ref[:, 3:4])
            scale_r, shift_r = _scale_shift(
                stacc_ref[:, 2:3], stacc_ref[:, 3:4], total,
                gb_ref[:, 4:5], gb_ref[:, 5:6])
            sscr_ref[:, 2:3] = scale2
            sscr_ref[:, 3:4] = shift2
            sscr_ref[:, 4:5] = scale_r
            sscr_ref[:, 5:6] = shift_r

        @pl.when(g >= 2 * G)
        def _phase_c():
            smp = g - 2 * G
            ssr = jnp.transpose(sscr_ref[:, 2:6])
            scale2, shift2 = ssr[0:1, :], ssr[1:2, :]
            scale_r, shift_r = ssr[2:3, :], ssr[3:4, :]
            y2t = jnp.transpose(ystore_ref[pl.ds(smp * C_out, C_out), :])
            xbn = y2t.astype(F32) * scale2 + shift2
            act = x_ref[pl.ds(smp % B, 1)][0].astype(BF16)
            rt = jax.lax.dot_general(
                act, wa_ref[...], (((0,), (1,)), ((), ())),
                preferred_element_type=F32)
            out_ref[0] = jnp.maximum(xbn + (rt * scale_r + shift_r), 0.0)
            xbn_ref[0] = xbn
    return body


def kernel(x, w1, w2, wa, g1, b1, g2, b2, ga, ba):
    N, C_in, D, H, W = x.shape
    C_out = w1.shape[0]
    L = D * H * W
    HW = H * W
    Lw = 2 * L + 2 * HW
    B = 2
    G = N // B

    x_cf = x.reshape(N, C_in, L)
    w13 = w1.reshape(C_out, 3, 9 * C_in).swapaxes(0, 1).astype(BF16)
    w23 = w2.reshape(C_out, 3, 9 * C_out).swapaxes(0, 1).astype(BF16)
    wab = wa.astype(BF16)
    gb = jnp.stack([g1, b1, g2, b2, ga, ba], axis=1)

    def x_idx(g):
        return (jnp.where(g < G, g, jnp.where(g < 2 * G, G - 1,
                                              (g - 2 * G) // B)), 0, 0)

    def out_idx(g):
        return (jnp.maximum(g - 2 * G, 0), 0, 0)

    out_nlc, xbn_nlc = pl.pallas_call(
        _make_body(N, C_in, C_out, D, H, W),
        out_shape=(jax.ShapeDtypeStruct((N, L, C_out), F32),
                   jax.ShapeDtypeStruct((N, L, C_out), F32)),
        grid=(4 * G,),
        in_specs=[pl.BlockSpec((B, C_in, L), x_idx),
                  pl.BlockSpec((3, C_out, 9 * C_in), lambda g: (0, 0, 0)),
                  pl.BlockSpec((3, C_out, 9 * C_out), lambda g: (0, 0, 0)),
                  pl.BlockSpec((C_out, C_in), lambda g: (0, 0)),
                  pl.BlockSpec((C_out, 6), lambda g: (0, 0))],
        out_specs=(pl.BlockSpec((1, L, C_out), out_idx),
                   pl.BlockSpec((1, L, C_out), out_idx)),
        scratch_shapes=[pltpu.VMEM((N * C_out, L), BF16),
                        pltpu.VMEM((C_out, 2 * PADL + Lw), BF16),
                        pltpu.VMEM((9 * C_out, 2 * HW + Lw), BF16),
                        pltpu.VMEM((C_out, 8), F32),
                        pltpu.VMEM((C_out, 8), F32)],
        compiler_params=pltpu.CompilerParams(
            dimension_semantics=("arbitrary",)),
    )(x_cf, w13, w23, wab, gb)

    out = out_nlc.reshape(N, D, H, W, C_out).transpose(0, 4, 1, 2, 3)
    xbn = xbn_nlc.reshape(N, D, H, W, C_out).transpose(0, 4, 1, 2, 3)
    return (out, xbn)
